# baseline (speedup 1.0000x reference)
import functools

import jax
import jax.numpy as jnp
from jax import lax
from jax.experimental import pallas as pl
from jax.experimental.pallas import tpu as pltpu
from jax.experimental.pallas import tpu_sc as plsc

def _bcast_lane(vec, lane):
    idx = jnp.full((vec.shape[0], 1), lane, jnp.int32)
    return lax.gather(
        vec, idx,
        lax.GatherDimensionNumbers(
            offset_dims=(), collapsed_slice_dims=(0,), start_index_map=(0,)),
        (1,), mode=lax.GatherScatterMode.PROMISE_IN_BOUNDS)


NC = 2
NS = 16
NW = NC * NS
LANES = 16
EBLK = 128


def _mm_body(x_ref, w_ref, o_ref, *, relu):
    x = x_ref[...]
    if relu:
        x = jnp.maximum(x, 0.0)
    o_ref[...] = lax.dot_general(
        x, w_ref[...], (((1,), (1,)), ((), ())),
        preferred_element_type=jnp.float32)


def _matmul(x, w, relu):
    m, k = x.shape
    o = w.shape[0]
    bm = 512
    return pl.pallas_call(
        functools.partial(_mm_body, relu=relu),
        grid=(pl.cdiv(m, bm),),
        in_specs=[
            pl.BlockSpec((bm, k), lambda i: (i, 0)),
            pl.BlockSpec((o, k), lambda i: (0, 0)),
        ],
        out_specs=pl.BlockSpec((bm, o), lambda i: (i, 0)),
        out_shape=jax.ShapeDtypeStruct((m, o), jnp.float32),
    )(x, w)


def _spmm_sc(b_mat, col, vals, row, bounds, n_nodes, rows_per):
    d = b_mat.shape[1]
    nj = d // LANES
    npad = NW * rows_per
    mesh = plsc.VectorSubcoreMesh(
        core_axis_name="c", subcore_axis_name="s",
        num_cores=NC, num_subcores=NS)

    def body(b_hbm, col_hbm, vals_hbm, row_hbm, bounds_hbm, out_hbm,
             bounds_v, colv, rowv, valv, rowsv, outv, sem):
        cid = lax.axis_index("c")
        sid = lax.axis_index("s")
        wid = sid * NC + cid
        off = pl.multiple_of(wid * LANES, 8)
        pltpu.sync_copy(bounds_hbm.at[pl.ds(off, LANES)], bounds_v)
        bvec = bounds_v[pl.ds(0, LANES)]
        e_lo = bvec[0]
        e_hi = bvec[1]
        r_lo = wid * rows_per
        e0 = e_lo - lax.rem(e_lo, 8)
        nblk = lax.div(e_hi - e0 + (EBLK - 1), EBLK)
        rp_u = jnp.uint32(rows_per)

        zeros16 = jnp.zeros((LANES,), jnp.float32)

        def zrow(i, c):
            outv[pl.ds(i * LANES, LANES)] = zeros16
            return c

        lax.fori_loop(0, rows_per * nj, zrow, 0)

        trash = r_lo + rows_per

        def clamped_base(prev):
            lu = prev - r_lo
            return jnp.where(lu.astype(jnp.uint32) < rp_u,
                             lu, rows_per) * d

        def blk_body(b, carry):
            eb = pl.multiple_of(e0 + b * EBLK, 8)
            pltpu.sync_copy(col_hbm.at[pl.ds(eb, EBLK)], colv)
            pltpu.sync_copy(row_hbm.at[pl.ds(eb, EBLK)], rowv)
            pltpu.sync_copy(vals_hbm.at[pl.ds(eb, EBLK)], valv)
            pltpu.async_copy(b_hbm.at[colv], rowsv, sem).wait()

            @plsc.parallel_loop(0, EBLK // LANES, carry=carry)
            def grp_body(g, gcarry):
                rv = rowv[pl.ds(g * LANES, LANES)]
                vv = valv[pl.ds(g * LANES, LANES)]
                for lane in range(LANES):
                    prev = gcarry[0]
                    acc = gcarry[1:]
                    e = g * LANES + lane
                    r = rv[lane]
                    v = vv[lane]
                    flush = r != prev

                    @pl.when(flush)
                    def _(prev=prev, acc=acc):
                        base = clamped_base(prev)
                        for j in range(nj):
                            outv[pl.ds(base + j * LANES, LANES)] = acc[j]

                    newacc = tuple(
                        jnp.where(flush, 0.0, acc[j])
                        + v * rowsv[e, pl.ds(j * LANES, LANES)]
                        for j in range(nj))
                    gcarry = (r,) + newacc
                return gcarry

            return grp_body

        init = (trash,) + tuple(jnp.zeros((LANES,), jnp.float32)
                                for _ in range(nj))
        final = lax.fori_loop(0, nblk, blk_body, init)
        fbase = clamped_base(final[0])
        for j in range(nj):
            outv[pl.ds(fbase + j * LANES, LANES)] = final[1 + j]
        pltpu.sync_copy(outv.at[pl.ds(0, rows_per * d)],
                        out_hbm.at[pl.ds(r_lo * d, rows_per * d)])

    k = pl.kernel(
        body,
        out_type=jax.ShapeDtypeStruct((npad * d,), jnp.float32),
        mesh=mesh,
        scratch_types=[
            pltpu.VMEM((LANES,), jnp.int32),
            pltpu.VMEM((EBLK,), jnp.int32),
            pltpu.VMEM((EBLK,), jnp.int32),
            pltpu.VMEM((EBLK,), jnp.float32),
            pltpu.VMEM((EBLK, d), jnp.float32),
            pltpu.VMEM(((rows_per + 1) * d,), jnp.float32),
            pltpu.SemaphoreType.DMA,
        ],
    )
    return k(b_mat, col, vals, row, bounds).reshape(npad, d)


def kernel(X, W1, W2, vals, row, col):
    n, _ = X.shape
    e = row.shape[0]
    rows_per = -(-n // (NW * 8)) * 8

    r_bounds = jnp.minimum(jnp.arange(NW + 1, dtype=jnp.int32) * rows_per, n)
    bnd = jnp.searchsorted(row, r_bounds, side="left").astype(jnp.int32)
    bounds = jnp.zeros((NW, 16), jnp.int32)
    bounds = bounds.at[:, 0].set(bnd[:NW]).at[:, 1].set(bnd[1:]).reshape(-1)
    pad = EBLK + 8
    colp = jnp.concatenate([col, jnp.zeros((pad,), col.dtype)])
    rowp = jnp.concatenate([row, jnp.full((pad,), n, row.dtype)])
    valsp = jnp.concatenate([vals, jnp.zeros((pad,), vals.dtype)])

    h = _matmul(X, W1, relu=False)
    h = _spmm_sc(h, colp, valsp, rowp, bounds, n, rows_per)[:n]
    h = _matmul(h, W2, relu=True)
    out = _spmm_sc(h, colp, valsp, rowp, bounds, n, rows_per)[:n]
    return out


# speedup vs baseline: 1.0014x; 1.0014x over previous
import functools

import jax
import jax.numpy as jnp
from jax import lax
from jax.experimental import pallas as pl
from jax.experimental.pallas import tpu as pltpu
from jax.experimental.pallas import tpu_sc as plsc

NC = 2
NS = 16
NW = NC * NS
LANES = 16
EBLK = 128


def _mm_body(x_ref, w_ref, o_ref, *, relu):
    x = x_ref[...]
    if relu:
        x = jnp.maximum(x, 0.0)
    o_ref[...] = lax.dot_general(
        x, w_ref[...], (((1,), (1,)), ((), ())),
        preferred_element_type=jnp.float32)


def _matmul(x, w, relu):
    m, k = x.shape
    o = w.shape[0]
    bm = 512
    return pl.pallas_call(
        functools.partial(_mm_body, relu=relu),
        grid=(pl.cdiv(m, bm),),
        in_specs=[
            pl.BlockSpec((bm, k), lambda i: (i, 0)),
            pl.BlockSpec((o, k), lambda i: (0, 0)),
        ],
        out_specs=pl.BlockSpec((bm, o), lambda i: (i, 0)),
        out_shape=jax.ShapeDtypeStruct((m, o), jnp.float32),
    )(x, w)


def _spmm_sc(b_mat, col, vals, row, bounds, n_nodes, rows_per):
    d = b_mat.shape[1]
    nj = d // LANES
    npad = NW * rows_per
    mesh = plsc.VectorSubcoreMesh(
        core_axis_name="c", subcore_axis_name="s",
        num_cores=NC, num_subcores=NS)

    def body(b_hbm, col_hbm, vals_hbm, row_hbm, bounds_hbm, out_hbm,
             bounds_v, colv, rowv, valv, rowsv, outv, sem):
        cid = lax.axis_index("c")
        sid = lax.axis_index("s")
        wid = sid * NC + cid
        off = pl.multiple_of(wid * LANES, 8)
        pltpu.sync_copy(bounds_hbm.at[pl.ds(off, LANES)], bounds_v)
        bvec = bounds_v[pl.ds(0, LANES)]
        e_lo = bvec[0]
        e_hi = bvec[1]
        r_lo = wid * rows_per
        e0 = e_lo - lax.rem(e_lo, 8)
        nblk = lax.div(e_hi - e0 + (EBLK - 1), EBLK)
        rp_u = jnp.uint32(rows_per)

        zeros16 = jnp.zeros((LANES,), jnp.float32)

        def zrow(i, c):
            outv[pl.ds(i * LANES, LANES)] = zeros16
            return c

        lax.fori_loop(0, rows_per * nj, zrow, 0)

        trash = r_lo + rows_per

        def clamped_base(prev):
            lu = prev - r_lo
            return jnp.where(lu.astype(jnp.uint32) < rp_u,
                             lu, rows_per) * d

        def blk_body(b, carry):
            eb = pl.multiple_of(e0 + b * EBLK, 8)
            pltpu.sync_copy(col_hbm.at[pl.ds(eb, EBLK)], colv)
            pltpu.sync_copy(row_hbm.at[pl.ds(eb, EBLK)], rowv)
            pltpu.sync_copy(vals_hbm.at[pl.ds(eb, EBLK)], valv)
            pltpu.async_copy(b_hbm.at[colv], rowsv, sem).wait()

            @plsc.parallel_loop(0, EBLK // LANES, carry=carry)
            def grp_body(g, gcarry):
                rv = rowv[pl.ds(g * LANES, LANES)]
                vv = valv[pl.ds(g * LANES, LANES)]
                for lane in range(LANES):
                    prev = gcarry[0]
                    acc = gcarry[1:]
                    e = g * LANES + lane
                    r = rv[lane]
                    v = vv[lane]
                    flush = r != prev

                    @pl.when(flush)
                    def _(prev=prev, acc=acc):
                        base = clamped_base(prev)
                        for j in range(nj):
                            outv[pl.ds(base + j * LANES, LANES)] = acc[j]

                    newacc = tuple(
                        jnp.where(flush, 0.0, acc[j])
                        + v * rowsv[e, pl.ds(j * LANES, LANES)]
                        for j in range(nj))
                    gcarry = (r,) + newacc
                return gcarry

            return grp_body

        init = (trash,) + tuple(jnp.zeros((LANES,), jnp.float32)
                                for _ in range(nj))
        final = lax.fori_loop(0, nblk, blk_body, init)
        fbase = clamped_base(final[0])
        for j in range(nj):
            outv[pl.ds(fbase + j * LANES, LANES)] = final[1 + j]
        pltpu.sync_copy(outv.at[pl.ds(0, rows_per * d)],
                        out_hbm.at[pl.ds(r_lo * d, rows_per * d)])

    k = pl.kernel(
        body,
        out_type=jax.ShapeDtypeStruct((npad * d,), jnp.float32),
        mesh=mesh,
        scratch_types=[
            pltpu.VMEM((LANES,), jnp.int32),
            pltpu.VMEM((EBLK,), jnp.int32),
            pltpu.VMEM((EBLK,), jnp.int32),
            pltpu.VMEM((EBLK,), jnp.float32),
            pltpu.VMEM((EBLK, d), jnp.float32),
            pltpu.VMEM(((rows_per + 1) * d,), jnp.float32),
            pltpu.SemaphoreType.DMA,
        ],
    )
    return k(b_mat, col, vals, row, bounds).reshape(npad, d)


def kernel(X, W1, W2, vals, row, col):
    n, _ = X.shape
    e = row.shape[0]
    rows_per = -(-n // (NW * 8)) * 8

    r_bounds = jnp.minimum(jnp.arange(NW + 1, dtype=jnp.int32) * rows_per, n)
    bnd = jnp.searchsorted(row, r_bounds, side="left").astype(jnp.int32)
    bounds = jnp.zeros((NW, 16), jnp.int32)
    bounds = bounds.at[:, 0].set(bnd[:NW]).at[:, 1].set(bnd[1:]).reshape(-1)
    pad = EBLK + 8
    colp = jnp.concatenate([col, jnp.zeros((pad,), col.dtype)])
    rowp = jnp.concatenate([row, jnp.full((pad,), n, row.dtype)])
    valsp = jnp.concatenate([vals, jnp.zeros((pad,), vals.dtype)])

    h = _matmul(X, W1, relu=False)
    h = _spmm_sc(h, colp, valsp, rowp, bounds, n, rows_per)[:n]
    h = _matmul(h, W2, relu=True)
    out = _spmm_sc(h, colp, valsp, rowp, bounds, n, rows_per)[:n]
    return out
